# baseline (speedup 1.0000x reference)
import functools

import jax
import jax.numpy as jnp
from jax import lax
from jax.experimental import pallas as pl
from jax.experimental.pallas import tpu as pltpu

EMBED = 64
SIZES = (512, 1024, 2048, 4096, 8192)
OFFS = (0, 512, 1536, 3584, 7680)
TOTAL = 15872
NTOK = 32768
BT = 256
BC = 512


CHUNKW = (512, 1024, 2048, 2048, 4096)


def _bf16_rtne(x):
    return x.astype(jnp.bfloat16).astype(jnp.float32)


def _argmin_body(x_ref, dall_ref, d2_ref, idx_ref):
    xb = x_ref[...]
    f2 = jnp.sum(xb * xb, axis=1, keepdims=True)
    xb16 = xb.astype(jnp.bfloat16)
    cols = []
    for k in range(len(SIZES)):
        run_v = None
        run_i = None
        for c0 in range(OFFS[k], OFFS[k] + SIZES[k], CHUNKW[k]):
            cm = None
            ci = None
            for c in range(c0, c0 + CHUNKW[k], BC):
                db = dall_ref[:, c:c + BC]
                sim = jnp.dot(xb16, db, preferred_element_type=jnp.float32)
                dist = (f2 + d2_ref[:, c:c + BC]) - 2.0 * sim
                bm = jnp.min(dist, axis=1, keepdims=True)
                ii = lax.broadcasted_iota(jnp.int32, (BT, BC), 1) + c
                barg = jnp.min(jnp.where(dist == bm, ii, jnp.int32(2**30)),
                               axis=1, keepdims=True)
                if cm is None:
                    cm, ci = bm, barg
                else:
                    upd = bm < cm
                    ci = jnp.where(upd, barg, ci)
                    cm = jnp.where(upd, bm, cm)
            if run_v is None:
                run_v, run_i = _bf16_rtne(cm), ci
            else:
                upd = cm < run_v
                run_i = jnp.where(upd, ci, run_i)
                run_v = _bf16_rtne(jnp.where(upd, cm, run_v))
        cols.append(run_i)
    idx_ref[...] = jnp.concatenate(cols, axis=1)


def _compute_indices(flat, dall16, d2):
    grid = NTOK // BT
    return pl.pallas_call(
        _argmin_body,
        grid=(grid,),
        in_specs=[
            pl.BlockSpec((BT, EMBED), lambda i: (i, 0)),
            pl.BlockSpec((EMBED, TOTAL), lambda i: (0, 0)),
            pl.BlockSpec((1, TOTAL), lambda i: (0, 0)),
        ],
        out_specs=pl.BlockSpec((BT, len(SIZES)), lambda i: (i, 0)),
        out_shape=jax.ShapeDtypeStruct((NTOK, len(SIZES)), jnp.int32),
    )(flat, dall16, d2)


def kernel(x, alpha_vq, dict_9, dict_10, dict_11, dict_12, dict_13):
    dicts = [dict_9, dict_10, dict_11, dict_12, dict_13]
    alpha = jax.nn.softmax(alpha_vq)
    flat = x.reshape(-1, EMBED)

    dall = jnp.concatenate(dicts, axis=1)
    d2 = jnp.sum(dall ** 2, axis=0, keepdims=True)

    gidx = _compute_indices(flat, dall.astype(jnp.bfloat16), d2)

    tab = dall.T * jnp.repeat(alpha, jnp.array(SIZES), total_repeat_length=TOTAL)[:, None]
    wq = jnp.sum(jnp.take(tab, gidx, axis=0), axis=1)
    return wq.reshape(x.shape)


# speedup vs baseline: 1.5986x; 1.5986x over previous
import functools

import jax
import jax.numpy as jnp
from jax import lax
from jax.experimental import pallas as pl
from jax.experimental.pallas import tpu as pltpu
from jax.experimental.pallas import tpu_sc as plsc

EMBED = 64
SIZES = (512, 1024, 2048, 4096, 8192)
OFFS = (0, 512, 1536, 3584, 7680)
TOTAL = 15872
NTOK = 32768
BT = 256
BC = 512


CHUNKW = (512, 1024, 2048, 2048, 4096)


def _bf16_rtne(x):
    return x.astype(jnp.bfloat16).astype(jnp.float32)


def _argmin_body(x_ref, dall_ref, d2_ref, idx_ref):
    xb = x_ref[...]
    f2 = jnp.sum(xb * xb, axis=1, keepdims=True)
    xb16 = xb.astype(jnp.bfloat16)
    cols = []
    for k in range(len(SIZES)):
        run_v = None
        run_i = None
        for c0 in range(OFFS[k], OFFS[k] + SIZES[k], CHUNKW[k]):
            cm = None
            ci = None
            for c in range(c0, c0 + CHUNKW[k], BC):
                db = dall_ref[:, c:c + BC]
                sim = jnp.dot(xb16, db, preferred_element_type=jnp.float32)
                dist = (f2 + d2_ref[:, c:c + BC]) - 2.0 * sim
                bm = jnp.min(dist, axis=1, keepdims=True)
                ii = lax.broadcasted_iota(jnp.int32, (BT, BC), 1) + c
                barg = jnp.min(jnp.where(dist == bm, ii, jnp.int32(2**30)),
                               axis=1, keepdims=True)
                if cm is None:
                    cm, ci = bm, barg
                else:
                    upd = bm < cm
                    ci = jnp.where(upd, barg, ci)
                    cm = jnp.where(upd, bm, cm)
            if run_v is None:
                run_v, run_i = _bf16_rtne(cm), ci
            else:
                upd = cm < run_v
                run_i = jnp.where(upd, ci, run_i)
                run_v = _bf16_rtne(jnp.where(upd, cm, run_v))
        cols.append(run_i)
    idx_ref[...] = jnp.concatenate(cols, axis=1)


def _compute_indices(flat, dall16, d2):
    grid = NTOK // BT
    return pl.pallas_call(
        _argmin_body,
        grid=(grid,),
        in_specs=[
            pl.BlockSpec((BT, EMBED), lambda i: (i, 0)),
            pl.BlockSpec((EMBED, TOTAL), lambda i: (0, 0)),
            pl.BlockSpec((1, TOTAL), lambda i: (0, 0)),
        ],
        out_specs=pl.BlockSpec((BT, len(SIZES)), lambda i: (i, 0)),
        out_shape=jax.ShapeDtypeStruct((NTOK, len(SIZES)), jnp.int32),
    )(flat, dall16, d2)


def kernel(x, alpha_vq, dict_9, dict_10, dict_11, dict_12, dict_13):
    dicts = [dict_9, dict_10, dict_11, dict_12, dict_13]
    alpha = jax.nn.softmax(alpha_vq)
    flat = x.reshape(-1, EMBED)

    dall = jnp.concatenate(dicts, axis=1)
    d2 = jnp.sum(dall ** 2, axis=0, keepdims=True)

    gidx = _compute_indices(flat, dall.astype(jnp.bfloat16), d2)

    tab = dall.T * jnp.repeat(alpha, jnp.array(SIZES), total_repeat_length=TOTAL)[:, None]
    tabp = jnp.pad(tab, ((0, 0), (0, 128 - EMBED)))
    idx2d = gidx.reshape(NTOK * len(SIZES) // 128, 128)
    wq = _sc_gather(tabp, idx2d)
    return wq.reshape(x.shape)


NW = 32
TPW = NTOK // NW
CH = 128
NCHUNK = TPW // CH


def _sc_gather(tab, idx2d):
    mesh = plsc.VectorSubcoreMesh(core_axis_name="c", subcore_axis_name="s")

    @functools.partial(
        pl.kernel,
        out_type=jax.ShapeDtypeStruct((NTOK, EMBED), jnp.float32),
        mesh=mesh,
        scratch_types=[
            pltpu.VMEM((128,), jnp.int32),
            pltpu.VMEM((128,), jnp.int32),
            pltpu.VMEM((128,), jnp.int32),
            pltpu.VMEM((128,), jnp.int32),
            pltpu.VMEM((128,), jnp.int32),
            pltpu.VMEM((5 * CH, 128), jnp.float32),
            pltpu.VMEM((CH, EMBED), jnp.float32),
            pltpu.SemaphoreType.DMA,
        ],
    )
    def body(tab_hbm, idx_hbm, out_hbm, i0, i1, i2, i3, i4, rows_v, acc_v, sem):
        wid = lax.axis_index("s") * 2 + lax.axis_index("c")
        ibufs = (i0, i1, i2, i3, i4)
        for ch in range(NCHUNK):
            row0 = wid * (TPW * 5 // 128) + ch * 5
            t0 = wid * TPW + ch * CH
            for g in range(5):
                pltpu.sync_copy(idx_hbm.at[row0 + g], ibufs[g])
            handles = [
                pltpu.async_copy(tab_hbm.at[ibufs[g]],
                                 rows_v.at[pl.ds(g * 128, 128)], sem)
                for g in range(5)
            ]
            for h in handles:
                h.wait()

            def tok(t, _):
                t5 = t * 5
                for c in range(EMBED // 16):
                    s = pl.ds(c * 16, 16)
                    acc_v[t, s] = ((rows_v[t5, s] + rows_v[t5 + 1, s])
                                   + (rows_v[t5 + 2, s] + rows_v[t5 + 3, s])
                                   + rows_v[t5 + 4, s])
                return 0
            lax.fori_loop(0, CH, tok, 0)
            pltpu.sync_copy(acc_v, out_hbm.at[pl.ds(t0, CH)])

    return body(tab, idx2d)
